# Initial kernel scaffold; baseline (speedup 1.0000x reference)
import functools

import jax
import jax.numpy as jnp
from jax import lax
from jax.experimental import pallas as pl
from jax.experimental.pallas import tpu as pltpu
from jax.experimental.pallas import tpu_sc as plsc

N_NODES = 10000
D = 256
L = 16
DC = D // L
NW = 32
CHUNK = 64
GPC = CHUNK // L


def _make_sc_kernel(e_pad: int):
    epw = e_pad // NW
    nchunk = epw // CHUNK
    mesh = plsc.VectorSubcoreMesh(core_axis_name="c", subcore_axis_name="s")

    @functools.partial(
        pl.kernel,
        mesh=mesh,
        out_type=jax.ShapeDtypeStruct((e_pad,), jnp.float32),
        scratch_types=[
            pltpu.VMEM((epw,), jnp.int32),
            pltpu.VMEM((epw,), jnp.int32),
            pltpu.VMEM((CHUNK, D), jnp.float32),
            pltpu.VMEM((CHUNK, D), jnp.float32),
            pltpu.VMEM((L, L), jnp.float32),
            pltpu.VMEM((epw,), jnp.float32),
            pltpu.VMEM((D,), jnp.float32),
            pltpu.VMEM((L,), jnp.float32),
            pltpu.SemaphoreType.DMA,
            pltpu.SemaphoreType.DMA,
        ],
    )
    def sc_kernel(h_hbm, src_hbm, dst_hbm, w_hbm, b_hbm, out_hbm,
                  sidx_v, didx_v, u_buf, v_buf, red_v, out_v, w_v, b_v,
                  sem_u, sem_v):
        wid = lax.axis_index("s") * 2 + lax.axis_index("c")
        base = wid * epw
        pltpu.sync_copy(src_hbm.at[pl.ds(base, epw)], sidx_v)
        pltpu.sync_copy(dst_hbm.at[pl.ds(base, epw)], didx_v)
        pltpu.sync_copy(w_hbm, w_v)
        pltpu.sync_copy(b_hbm, b_v)
        w_regs = [w_v[pl.ds(c * L, L)] for c in range(DC)]
        b_reg = b_v[...]
        iota = lax.iota(jnp.int32, L)

        def chunk_body(k, carry):
            cb = k * CHUNK
            cp_u = pltpu.async_copy(
                h_hbm.at[sidx_v.at[pl.ds(cb, CHUNK)]], u_buf, sem_u)
            cp_v = pltpu.async_copy(
                h_hbm.at[didx_v.at[pl.ds(cb, CHUNK)]], v_buf, sem_v)
            cp_u.wait()
            cp_v.wait()

            def group_body(g, carry2):
                e0 = g * L
                accs = [None] * L
                for c in range(DC):
                    wc = w_regs[c]
                    for e in range(L):
                        u = u_buf[e0 + e, pl.ds(c * L, L)]
                        v = v_buf[e0 + e, pl.ds(c * L, L)]
                        p = u * (v * wc)
                        accs[e] = p if c == 0 else accs[e] + p
                for e in range(L):
                    red_v[e, :] = accs[e]
                tot = b_reg
                for dcol in range(L):
                    col = plsc.load_gather(
                        red_v, [iota, jnp.full((L,), dcol, jnp.int32)])
                    tot = tot + col
                out_v[pl.ds(cb + e0, L)] = tot
                return carry2

            lax.fori_loop(0, GPC, group_body, 0)
            return carry

        lax.fori_loop(0, nchunk, chunk_body, 0)
        pltpu.sync_copy(out_v, out_hbm.at[pl.ds(base, epw)])

    return sc_kernel


def kernel(h, edge_index, W1_w, W1_b):
    e = edge_index.shape[1]
    e_pad = ((e + NW * CHUNK - 1) // (NW * CHUNK)) * (NW * CHUNK)
    src = edge_index[0].astype(jnp.int32)
    dst = edge_index[1].astype(jnp.int32)
    pad = e_pad - e
    if pad:
        src = jnp.concatenate([src, jnp.zeros((pad,), jnp.int32)])
        dst = jnp.concatenate([dst, jnp.zeros((pad,), jnp.int32)])
    w = W1_w.reshape(D).astype(jnp.float32)
    bvec = jnp.broadcast_to(W1_b.reshape(1).astype(jnp.float32), (L,))
    out = _make_sc_kernel(e_pad)(h.astype(jnp.float32), src, dst, w, bvec)
    return out[:e]


# speedup vs baseline: 1.1197x; 1.1197x over previous
import functools

import jax
import jax.numpy as jnp
from jax import lax
from jax.experimental import pallas as pl
from jax.experimental.pallas import tpu as pltpu
from jax.experimental.pallas import tpu_sc as plsc

N_NODES = 10000
D = 256
L = 16
DC = D // L
NW = 32
CHUNK = 64
GPC = CHUNK // L


def _make_sc_kernel(e_pad: int):
    epw = e_pad // NW
    nchunk = epw // CHUNK
    mesh = plsc.VectorSubcoreMesh(core_axis_name="c", subcore_axis_name="s")

    @functools.partial(
        pl.kernel,
        mesh=mesh,
        out_type=jax.ShapeDtypeStruct((e_pad,), jnp.float32),
        compiler_params=pltpu.CompilerParams(needs_layout_passes=False),
        scratch_types=[
            pltpu.VMEM((epw,), jnp.int32),
            pltpu.VMEM((epw,), jnp.int32),
            pltpu.VMEM((CHUNK, D), jnp.float32),
            pltpu.VMEM((CHUNK, D), jnp.float32),
            pltpu.VMEM((epw,), jnp.float32),
            pltpu.VMEM((D,), jnp.float32),
            pltpu.VMEM((L,), jnp.float32),
            pltpu.SemaphoreType.DMA,
            pltpu.SemaphoreType.DMA,
        ],
    )
    def sc_kernel(h_hbm, src_hbm, dst_hbm, w_hbm, b_hbm, out_hbm,
                  sidx_v, didx_v, u_buf, v_buf, out_v, w_v, b_v,
                  sem_u, sem_v):
        wid = lax.axis_index("s") * 2 + lax.axis_index("c")
        base = wid * epw
        pltpu.sync_copy(src_hbm.at[pl.ds(base, epw)], sidx_v)
        pltpu.sync_copy(dst_hbm.at[pl.ds(base, epw)], didx_v)
        pltpu.sync_copy(w_hbm, w_v)
        pltpu.sync_copy(b_hbm, b_v)
        w_regs = [w_v[pl.ds(c * L, L)] for c in range(DC)]
        b_reg = b_v[...]
        iota = lax.iota(jnp.int32, L)
        lane_masks = [iota == e for e in range(L)]

        def chunk_body(k, carry):
            cb = k * CHUNK
            cp_u = pltpu.async_copy(
                h_hbm.at[sidx_v.at[pl.ds(cb, CHUNK)]], u_buf, sem_u)
            cp_v = pltpu.async_copy(
                h_hbm.at[didx_v.at[pl.ds(cb, CHUNK)]], v_buf, sem_v)
            cp_u.wait()
            cp_v.wait()

            def group_body(g, carry2):
                e0 = g * L
                accs = [None] * L
                for c in range(DC):
                    wc = w_regs[c]
                    for e in range(L):
                        u = u_buf[e0 + e, pl.ds(c * L, L)]
                        v = v_buf[e0 + e, pl.ds(c * L, L)]
                        p = u * (v * wc)
                        accs[e] = p if c == 0 else accs[e] + p
                tot = b_reg
                for e in range(L):
                    s = jnp.sum(accs[e])
                    tot = jnp.where(lane_masks[e],
                                    jnp.broadcast_to(s, (L,)), tot)
                out_v[pl.ds(cb + e0, L)] = tot + b_reg
                return carry2

            lax.fori_loop(0, GPC, group_body, 0)
            return carry

        lax.fori_loop(0, nchunk, chunk_body, 0)
        pltpu.sync_copy(out_v, out_hbm.at[pl.ds(base, epw)])

    return sc_kernel


def kernel(h, edge_index, W1_w, W1_b):
    e = edge_index.shape[1]
    e_pad = ((e + NW * CHUNK - 1) // (NW * CHUNK)) * (NW * CHUNK)
    src = edge_index[0].astype(jnp.int32)
    dst = edge_index[1].astype(jnp.int32)
    pad = e_pad - e
    if pad:
        src = jnp.concatenate([src, jnp.zeros((pad,), jnp.int32)])
        dst = jnp.concatenate([dst, jnp.zeros((pad,), jnp.int32)])
    w = W1_w.reshape(D).astype(jnp.float32)
    bvec = jnp.broadcast_to(W1_b.reshape(1).astype(jnp.float32), (L,))
    out = _make_sc_kernel(e_pad)(h.astype(jnp.float32), src, dst, w, bvec)
    return out[:e]


# speedup vs baseline: 1.9049x; 1.7013x over previous
import functools

import jax
import jax.numpy as jnp
from jax import lax
from jax.experimental import pallas as pl
from jax.experimental.pallas import tpu as pltpu
from jax.experimental.pallas import tpu_sc as plsc

N_NODES = 10000
D = 256
L = 16
DC = D // L
NW = 32
CHUNK = 64
GPC = CHUNK // L


def _make_sc_kernel(e_pad: int):
    epw = e_pad // NW
    nchunk = epw // CHUNK
    mesh = plsc.VectorSubcoreMesh(core_axis_name="c", subcore_axis_name="s")

    @functools.partial(
        pl.kernel,
        mesh=mesh,
        out_type=jax.ShapeDtypeStruct((e_pad,), jnp.float32),
        compiler_params=pltpu.CompilerParams(needs_layout_passes=False),
        scratch_types=[
            pltpu.VMEM((epw,), jnp.int32),
            pltpu.VMEM((epw,), jnp.int32),
            pltpu.VMEM((2, CHUNK, D), jnp.float32),
            pltpu.VMEM((2, CHUNK, D), jnp.float32),
            pltpu.VMEM((epw,), jnp.float32),
            pltpu.VMEM((D,), jnp.float32),
            pltpu.VMEM((L,), jnp.float32),
            pltpu.SemaphoreType.DMA,
            pltpu.SemaphoreType.DMA,
            pltpu.SemaphoreType.DMA,
            pltpu.SemaphoreType.DMA,
        ],
    )
    def sc_kernel(h_hbm, src_hbm, dst_hbm, w_hbm, b_hbm, out_hbm,
                  sidx_v, didx_v, u_buf, v_buf, out_v, w_v, b_v,
                  sem_u0, sem_u1, sem_v0, sem_v1):
        wid = lax.axis_index("s") * 2 + lax.axis_index("c")
        base = wid * epw
        pltpu.sync_copy(src_hbm.at[pl.ds(base, epw)], sidx_v)
        pltpu.sync_copy(dst_hbm.at[pl.ds(base, epw)], didx_v)
        pltpu.sync_copy(w_hbm, w_v)
        pltpu.sync_copy(b_hbm, b_v)
        w_regs = [w_v[pl.ds(c * L, L)] for c in range(DC)]
        b_reg = b_v[...]
        iota = lax.iota(jnp.int32, L)
        lane_masks = [iota == e for e in range(L)]

        sems = ((sem_u0, sem_v0), (sem_u1, sem_v1))

        def start(k, slot):
            cb = k * CHUNK
            cp_u = pltpu.async_copy(
                h_hbm.at[sidx_v.at[pl.ds(cb, CHUNK)]], u_buf.at[slot],
                sems[slot][0])
            cp_v = pltpu.async_copy(
                h_hbm.at[didx_v.at[pl.ds(cb, CHUNK)]], v_buf.at[slot],
                sems[slot][1])
            return cp_u, cp_v

        def compute(k, slot):
            cb = k * CHUNK

            def group_body(g, carry2):
                e0 = g * L
                accs = [None] * L
                for c in range(DC):
                    wc = w_regs[c]
                    for e in range(L):
                        u = u_buf[slot, e0 + e, pl.ds(c * L, L)]
                        v = v_buf[slot, e0 + e, pl.ds(c * L, L)]
                        p = u * (v * wc)
                        accs[e] = p if c == 0 else accs[e] + p
                tot = b_reg
                for e in range(L):
                    s = jnp.sum(accs[e])
                    tot = jnp.where(lane_masks[e],
                                    jnp.broadcast_to(s, (L,)), tot)
                out_v[pl.ds(cb + e0, L)] = tot + b_reg
                return carry2

            lax.fori_loop(0, GPC, group_body, 0)

        def wait(k, slot):
            cb = k * CHUNK
            pltpu.make_async_copy(
                h_hbm.at[sidx_v.at[pl.ds(cb, CHUNK)]], u_buf.at[slot],
                sems[slot][0]).wait()
            pltpu.make_async_copy(
                h_hbm.at[didx_v.at[pl.ds(cb, CHUNK)]], v_buf.at[slot],
                sems[slot][1]).wait()

        npair = nchunk // 2
        start(0, 0)

        def pair_body(p, carry):
            k0 = p * 2
            start(k0 + 1, 1)
            wait(k0, 0)
            compute(k0, 0)

            @pl.when(p < npair - 1)
            def _():
                start(k0 + 2, 0)

            wait(k0 + 1, 1)
            compute(k0 + 1, 1)
            return carry

        lax.fori_loop(0, npair, pair_body, 0)
        pltpu.sync_copy(out_v, out_hbm.at[pl.ds(base, epw)])

    return sc_kernel


def kernel(h, edge_index, W1_w, W1_b):
    e = edge_index.shape[1]
    e_pad = ((e + NW * CHUNK - 1) // (NW * CHUNK)) * (NW * CHUNK)
    src = edge_index[0].astype(jnp.int32)
    dst = edge_index[1].astype(jnp.int32)
    pad = e_pad - e
    if pad:
        src = jnp.concatenate([src, jnp.zeros((pad,), jnp.int32)])
        dst = jnp.concatenate([dst, jnp.zeros((pad,), jnp.int32)])
    w = W1_w.reshape(D).astype(jnp.float32)
    bvec = jnp.broadcast_to(W1_b.reshape(1).astype(jnp.float32), (L,))
    out = _make_sc_kernel(e_pad)(h.astype(jnp.float32), src, dst, w, bvec)
    return out[:e]
